# baseline (speedup 1.0000x reference)
import jax
import jax.numpy as jnp
from jax.experimental import pallas as pl
from jax.experimental.pallas import tpu as pltpu

DIM = 384
QK = 384
HEADS = 8
CH = DIM // HEADS
HP = 128
QP = HEADS * HP
NWIN = 7
P2 = NWIN * NWIN
WS = 8
W2 = WS * WS
TOPK = 4
SCALE = QK ** (-0.5)
N = 4
ACOLS = 3 * QP + DIM


def _qkv_kernel(x_ref, w_ref, b_ref, q_ref, k_ref, v_ref, vi_ref, qs_ref, ks_ref):
    xb = x_ref[0].reshape(WS * 56, DIM)
    acc = jnp.dot(xb, w_ref[...], preferred_element_type=jnp.float32)
    acc = acc + b_ref[...]
    vi_ref[0] = acc[:, 3 * QP:].reshape(WS, 56, DIM)
    acc3 = acc.reshape(WS, 56, ACOLS)
    for i in range(NWIN):
        blk = acc3[:, i * WS:(i + 1) * WS, :].reshape(W2, ACOLS)
        q_ref[0, i] = blk[:, :QP]
        k_ref[0, i] = blk[:, QP:2 * QP]
        v_ref[0, i] = blk[:, 2 * QP:3 * QP]
        qs_ref[0, i, 0] = jnp.sum(blk[:, :QP], axis=0)
        ks_ref[0, i, 0] = jnp.sum(blk[:, QP:2 * QP], axis=0)


def _route_kernel(qs_ref, ks_ref, idx_ref):
    qm = qs_ref[0][:, 0, :] * (1.0 / W2)
    km = ks_ref[0][:, 0, :] * (1.0 / W2)
    qc = jnp.concatenate([qm[:, h * HP:h * HP + CH] for h in range(HEADS)],
                         axis=1) * SCALE
    kc = jnp.concatenate([km[:, h * HP:h * HP + CH] for h in range(HEADS)],
                         axis=1)
    logits = jax.lax.dot_general(qc, kc, (((1,), (1,)), ((), ())),
                                 preferred_element_type=jnp.float32)
    col = jax.lax.broadcasted_iota(jnp.int32, (P2, 128), 1)
    lp = jnp.where(col < P2,
                   jnp.pad(logits, ((0, 0), (0, 128 - P2))),
                   jnp.float32(-1e30))
    cols = []
    for _ in range(TOPK):
        am = jnp.argmax(lp, axis=-1).astype(jnp.int32)
        cols.append(am[:, None])
        lp = jnp.where(col == am[:, None], jnp.float32(-1e30), lp)
    idx_ref[0, 0] = jnp.concatenate(cols, axis=1)


def _attn_kernel(idx_ref, q_ref, k_ref, v_ref, o_ref):
    n = pl.program_id(0)
    j = pl.program_id(1)
    base = (n * P2 + j * NWIN) * TOPK

    def stage_qk(i):
        q = q_ref[0, i] * SCALE
        iv = [idx_ref[base + i * TOPK + t] for t in range(TOPK)]
        kcat = jnp.concatenate([k_ref[0, t] for t in iv], axis=0)
        vcat = jnp.concatenate([v_ref[0, t] for t in iv], axis=0)
        ls = [jax.lax.dot_general(q[:, h * HP:(h + 1) * HP],
                                  kcat[:, h * HP:(h + 1) * HP],
                                  (((1,), (1,)), ((), ())),
                                  preferred_element_type=jnp.float32)
              for h in range(HEADS)]
        return ls, vcat

    def stage_m(ls):
        mm = ls[0]
        for l in ls[1:]:
            mm = jnp.maximum(mm, l)
        return jnp.max(mm, axis=-1, keepdims=True)

    def stage_out(i, ls, m, vcat):
        parts = []
        for h in range(HEADS):
            p = jnp.exp(ls[h] - m)
            oa = jnp.dot(p, vcat[:, h * HP:(h + 1) * HP],
                         preferred_element_type=jnp.float32)
            parts.append(oa[:, :CH] / oa[:, CH:CH + 1])
        ocat = jnp.concatenate(parts, axis=-1)
        o_ref[0, :, i * WS:(i + 1) * WS, :] = ocat.reshape(WS, WS, DIM)

    for ia, ib in ((0, 1), (2, 3), (4, 5)):
        lsa, vca = stage_qk(ia)
        lsb, vcb = stage_qk(ib)
        ma = stage_m(lsa)
        mb = stage_m(lsb)
        stage_out(ia, lsa, ma, vca)
        stage_out(ib, lsb, mb, vcb)
    ls, vc = stage_qk(6)
    stage_out(6, ls, stage_m(ls), vc)


def _tail_kernel(a_ref, v_ref, lw_ref, lb_ref, w_ref, b_ref, o_ref, scr):
    v = v_ref[0]
    scr[...] = jnp.zeros((56, 56, DIM), jnp.float32) + lb_ref[0]
    for dy in range(3):
        for dx in range(3):
            wv = lw_ref[dy * 3 + dx]
            oy0, oy1 = max(0, 1 - dy), 56 - max(0, dy - 1)
            ox0, ox1 = max(0, 1 - dx), 56 - max(0, dx - 1)
            iy0, iy1 = oy0 + dy - 1, oy1 + dy - 1
            ix0, ix1 = ox0 + dx - 1, ox1 + dx - 1
            scr[oy0:oy1, ox0:ox1, :] += v[iy0:iy1, ix0:ix1, :] * wv
    s = (a_ref[0] + scr[...]).reshape(56 * 56, DIM)
    out = jnp.dot(s, w_ref[...], preferred_element_type=jnp.float32) + b_ref[...]
    o_ref[0] = out.reshape(56, 56, DIM)


def _pad_heads(w):
    w3 = w.reshape(w.shape[:-1] + (HEADS, CH))
    pad = [(0, 0)] * (w3.ndim - 1) + [(0, HP - CH)]
    return jnp.pad(w3, pad).reshape(w.shape[:-1] + (QP,))


def kernel(x, qkv_w, qkv_b, wo_w, wo_b, lepe_w, lepe_b):
    wqkvT = qkv_w.T
    wbig = jnp.concatenate([
        _pad_heads(wqkvT[:, :QK]),
        _pad_heads(wqkvT[:, QK:2 * QK]),
        _pad_heads(wqkvT[:, 2 * QK:]),
        wqkvT[:, 2 * QK:],
    ], axis=1)
    ones_col = jnp.zeros((QP,), jnp.float32).at[
        jnp.arange(HEADS) * HP + CH].set(1.0)
    bbig = jnp.concatenate([
        _pad_heads(qkv_b[:QK]),
        _pad_heads(qkv_b[QK:2 * QK]),
        _pad_heads(qkv_b[2 * QK:]) + ones_col,
        qkv_b[2 * QK:],
    ])[None, :]
    woT = wo_w.T
    lw = lepe_w[:, 0].transpose(1, 2, 0).reshape(9, DIM)

    q, k, v, v_img, qs, ks = pl.pallas_call(
        _qkv_kernel,
        grid=(N, NWIN),
        in_specs=[
            pl.BlockSpec((1, WS, 56, DIM), lambda n, j: (n, j, 0, 0)),
            pl.BlockSpec((DIM, ACOLS), lambda n, j: (0, 0)),
            pl.BlockSpec((1, ACOLS), lambda n, j: (0, 0)),
        ],
        out_specs=[
            pl.BlockSpec((1, NWIN, W2, QP), lambda n, j: (n, j, 0, 0)),
            pl.BlockSpec((1, NWIN, W2, QP), lambda n, j: (n, j, 0, 0)),
            pl.BlockSpec((1, NWIN, W2, QP), lambda n, j: (n, j, 0, 0)),
            pl.BlockSpec((1, WS, 56, DIM), lambda n, j: (n, j, 0, 0)),
            pl.BlockSpec((1, NWIN, 1, QP), lambda n, j: (n, j, 0, 0)),
            pl.BlockSpec((1, NWIN, 1, QP), lambda n, j: (n, j, 0, 0)),
        ],
        out_shape=[
            jax.ShapeDtypeStruct((N, P2, W2, QP), jnp.float32),
            jax.ShapeDtypeStruct((N, P2, W2, QP), jnp.float32),
            jax.ShapeDtypeStruct((N, P2, W2, QP), jnp.float32),
            jax.ShapeDtypeStruct((N, 56, 56, DIM), jnp.float32),
            jax.ShapeDtypeStruct((N, P2, 1, QP), jnp.float32),
            jax.ShapeDtypeStruct((N, P2, 1, QP), jnp.float32),
        ],
    )(x, wbig, bbig)

    r_idx = pl.pallas_call(
        _route_kernel,
        grid=(N,),
        in_specs=[
            pl.BlockSpec((1, P2, 1, QP), lambda n: (n, 0, 0, 0)),
            pl.BlockSpec((1, P2, 1, QP), lambda n: (n, 0, 0, 0)),
        ],
        out_specs=pl.BlockSpec((1, 1, P2, TOPK), lambda n: (n, 0, 0, 0)),
        out_shape=jax.ShapeDtypeStruct((N, 1, P2, TOPK), jnp.int32),
    )(qs, ks)
    idx_flat = r_idx.reshape(N * P2 * TOPK)

    attn_img = pl.pallas_call(
        _attn_kernel,
        grid_spec=pltpu.PrefetchScalarGridSpec(
            num_scalar_prefetch=1,
            grid=(N, NWIN),
            in_specs=[
                pl.BlockSpec((1, NWIN, W2, QP), lambda n, j, idx: (n, j, 0, 0)),
                pl.BlockSpec((1, P2, W2, QP), lambda n, j, idx: (n, 0, 0, 0)),
                pl.BlockSpec((1, P2, W2, QP), lambda n, j, idx: (n, 0, 0, 0)),
            ],
            out_specs=pl.BlockSpec((1, WS, 56, DIM),
                                   lambda n, j, idx: (n, j, 0, 0)),
        ),
        out_shape=jax.ShapeDtypeStruct((N, 56, 56, DIM), jnp.float32),
    )(idx_flat, q, k, v)

    out = pl.pallas_call(
        _tail_kernel,
        grid=(N,),
        in_specs=[
            pl.BlockSpec((1, 56, 56, DIM), lambda n: (n, 0, 0, 0)),
            pl.BlockSpec((1, 56, 56, DIM), lambda n: (n, 0, 0, 0)),
            pl.BlockSpec((9, DIM), lambda n: (0, 0)),
            pl.BlockSpec((1, DIM), lambda n: (0, 0)),
            pl.BlockSpec((DIM, DIM), lambda n: (0, 0)),
            pl.BlockSpec((1, DIM), lambda n: (0, 0)),
        ],
        out_specs=pl.BlockSpec((1, 56, 56, DIM), lambda n: (n, 0, 0, 0)),
        out_shape=jax.ShapeDtypeStruct((N, 56, 56, DIM), jnp.float32),
        scratch_shapes=[pltpu.VMEM((56, 56, DIM), jnp.float32)],
    )(attn_img, v_img, lw, lepe_b[None, :], woT, wo_b[None, :])

    return out
